# baseline (speedup 1.0000x reference)
import functools

import jax
import jax.numpy as jnp
from jax import lax
from jax.experimental import pallas as pl
from jax.experimental.pallas import tpu as pltpu
from jax.experimental.pallas import tpu_sc as plsc


def _make_lookup(B, L, V, D, interpret=False):
    NC, NS = 2, 16
    NW = NC * NS
    assert B % NW == 0
    seq_per_w = B // NW
    mesh = plsc.VectorSubcoreMesh(core_axis_name="c", subcore_axis_name="s",
                                  num_cores=NC, num_subcores=NS)

    @functools.partial(
        pl.kernel,
        out_type=jax.ShapeDtypeStruct((B * L, D), jnp.float32),
        mesh=mesh,
        scratch_types=[
            pltpu.VMEM((L,), jnp.int32),
            pltpu.VMEM((L, D), jnp.float32),
            pltpu.VMEM((L, D), jnp.float32),
            pltpu.SemaphoreType.DMA,
        ],
        interpret=interpret,
        compiler_params=pltpu.CompilerParams(use_tc_tiling_on_sc=False),
    )
    def lookup(ids_hbm, wte_hbm, wpe_hbm, out_hbm, idx_v, rows_v, wpe_v, sem):
        wid = lax.axis_index("s") * NC + lax.axis_index("c")

        pltpu.sync_copy(wpe_hbm, wpe_v)

        def seq_body(i, carry):
            base = (wid * seq_per_w + i) * L
            pltpu.sync_copy(ids_hbm.at[pl.ds(base, L)], idx_v)
            pltpu.async_copy(wte_hbm.at[idx_v], rows_v, sem).wait()

            def row_body(r, c2):
                for j in range(D // 16):
                    sl = pl.ds(j * 16, 16)
                    rows_v[r, sl] = rows_v[r, sl] + wpe_v[r, sl]
                return c2

            lax.fori_loop(0, L, row_body, 0)
            pltpu.sync_copy(rows_v, out_hbm.at[pl.ds(base, L)])
            return carry

        lax.fori_loop(0, seq_per_w, seq_body, 0)

    return lookup


def kernel(input_ids, wte_table, wpe_table):
    B, L = input_ids.shape
    V, D = wte_table.shape
    ids_flat = input_ids.reshape(B * L).astype(jnp.int32)
    wpe = wpe_table[:L]
    out = _make_lookup(B, L, V, D)(ids_flat, wte_table, wpe)
    return out.reshape(B, L, D)
